# baseline (speedup 1.0000x reference)
import functools

import jax
import jax.numpy as jnp
from jax import lax
from jax.experimental import pallas as pl
from jax.experimental.pallas import tpu as pltpu
from jax.experimental.pallas import tpu_sc as plsc

_D = 256
_B = 16
_N = 1024
_HW = 32
_CHUNK = 128
_NC = 2
_NS = 16
_NW = _NC * _NS
_TOK = _B * _N
_NBUF = 3
_CPW = 4


def _build_sc_gather():
    mesh = plsc.VectorSubcoreMesh(core_axis_name="c", subcore_axis_name="s")

    @functools.partial(
        pl.kernel,
        mesh=mesh,
        compiler_params=pltpu.CompilerParams(
            needs_layout_passes=False,
            disable_bounds_checks=True,
            disable_semaphore_checks=True,
        ),
        out_type=jax.ShapeDtypeStruct((_TOK, _D), jnp.float32),
        scratch_types=[
            pltpu.VMEM((8, _CHUNK), jnp.int32),
            pltpu.VMEM((_NBUF, _CHUNK, _D), jnp.float32),
            pltpu.SemaphoreType.DMA,
            pltpu.SemaphoreType.DMA,
            pltpu.SemaphoreType.DMA,
            pltpu.SemaphoreType.DMA,
            pltpu.SemaphoreType.DMA,
            pltpu.SemaphoreType.DMA,
        ],
    )
    def k(seq_hbm, emb_hbm, out_hbm, idx_v, g_v, sg0, sg1, sg2, sw0, sw1, sw2):
        wid = lax.axis_index("s") * _NC + lax.axis_index("c")
        grp = wid // 16
        j = (wid // 2) % 8
        h = wid % 2
        sg = [sg0, sg1, sg2]
        sw = [sw0, sw1, sw2]

        pltpu.sync_copy(
            seq_hbm.at[pl.ds(grp * 8, 8), pl.ds(j * _CHUNK, _CHUNK)], idx_v
        )

        def start_gather(c):
            return pltpu.async_copy(
                emb_hbm.at[idx_v.at[h * _CPW + c]],
                g_v.at[c % _NBUF],
                sg[c % _NBUF],
            )

        def start_write(c):
            b = grp * 8 + h * _CPW + c
            return pltpu.async_copy(
                g_v.at[c % _NBUF],
                out_hbm.at[pl.ds(b * _N + j * _CHUNK, _CHUNK)],
                sw[c % _NBUF],
            )

        gathers = [None] * _CPW
        writes = [None] * _CPW
        for c in range(min(_NBUF - 1, _CPW)):
            gathers[c] = start_gather(c)
        for c in range(_CPW):
            gathers[c].wait()
            writes[c] = start_write(c)
            nxt = c + _NBUF - 1
            if nxt < _CPW:
                prev = nxt - _NBUF
                if prev >= 0:
                    writes[prev].wait()
                gathers[nxt] = start_gather(nxt)
        for c in range(max(0, _CPW - _NBUF), _CPW):
            if writes[c] is not None:
                writes[c].wait()

    return k


_sc_gather = _build_sc_gather()


def kernel(seq, embedding):
    rows = _sc_gather(seq.astype(jnp.int32), embedding)
    out = rows.reshape(_B, _HW, _HW, _D)
    return jnp.transpose(out, (0, 3, 1, 2))
